# Initial kernel scaffold; baseline (speedup 1.0000x reference)
import functools

import jax
import jax.numpy as jnp
from jax import lax
from jax.experimental import pallas as pl
from jax.experimental.pallas import tpu as pltpu
from jax.experimental.pallas import tpu_sc as plsc

N = 10000
E = 320000
NCORES = 2
NSUB = 16
NPAD = 10240
RPT = NPAD // NSUB
EPT = E // NSUB
K = 80
NCH = EPT // K


def _make_sc_agg(F2, with_cnt):
    out_type = [jax.ShapeDtypeStruct((NCORES, NPAD, F2), jnp.float32)]
    if with_cnt:
        out_type.append(jax.ShapeDtypeStruct((NPAD,), jnp.float32))
    scratch = [
        pltpu.VMEM((K,), jnp.int32),
        pltpu.VMEM((K,), jnp.int32),
        pltpu.VMEM((K, F2), jnp.float32),
        pltpu.VMEM((K,), jnp.float32),
        pltpu.VMEM((RPT,), jnp.float32),
        pltpu.VMEM_SHARED((NPAD, F2), jnp.float32),
        pltpu.VMEM_SHARED((NPAD,), jnp.float32),
    ]
    mesh = plsc.VectorSubcoreMesh(core_axis_name="c", subcore_axis_name="s")

    def body(x_hbm, src_hbm, dst_hbm, zer_hbm, *rest):
        if with_cnt:
            agg_hbm, cnt_hbm = rest[0], rest[1]
            idxs, idxd, rows, ones_v, zc_v, acc, cntacc = rest[2:]
        else:
            agg_hbm = rest[0]
            idxs, idxd, rows, ones_v, zc_v, acc, cntacc = rest[1:]
        c = lax.axis_index("c")
        s = lax.axis_index("s")
        r0 = s * RPT

        pltpu.sync_copy(zer_hbm.at[pl.ds(r0, RPT)], acc.at[pl.ds(r0, RPT)])
        if with_cnt:
            for i in range(RPT // 16):
                zc_v[pl.ds(i * 16, 16)] = jnp.zeros((16,), jnp.float32)
            for i in range(K // 16):
                ones_v[pl.ds(i * 16, 16)] = jnp.ones((16,), jnp.float32)
            pltpu.sync_copy(zc_v, cntacc.at[pl.ds(r0, RPT)])
        plsc.subcore_barrier()

        def chunk(j, carry):
            base = s * EPT + j * K
            pltpu.sync_copy(src_hbm.at[pl.ds(base, K)], idxs)
            pltpu.sync_copy(dst_hbm.at[pl.ds(base, K)], idxd)
            pltpu.sync_copy(x_hbm.at[c].at[idxs], rows)
            pltpu.sync_copy(rows, acc.at[idxd], add=True)
            if with_cnt:
                @pl.when(c == 0)
                def _():
                    pltpu.sync_copy(ones_v, cntacc.at[idxd], add=True)
            return carry

        lax.fori_loop(0, NCH, chunk, 0)
        plsc.subcore_barrier()

        pltpu.sync_copy(acc.at[pl.ds(r0, RPT)], agg_hbm.at[c].at[pl.ds(r0, RPT)])
        if with_cnt:
            @pl.when(c == 0)
            def _():
                pltpu.sync_copy(cntacc.at[pl.ds(r0, RPT)], cnt_hbm.at[pl.ds(r0, RPT)])

    return pl.kernel(body, out_type=tuple(out_type), mesh=mesh,
                     scratch_types=scratch)


def _make_tc_layer(F2i, Fo, relu, split_out, R=500):
    F = 2 * F2i
    F2o = Fo // 2
    grid = (N // R,)
    in_specs = [
        pl.BlockSpec((NCORES, R, F2i), lambda i: (0, i, 0)),
        pl.BlockSpec((NCORES, R, F2i), lambda i: (0, i, 0)),
        pl.BlockSpec((R, 1), lambda i: (i, 0)),
        pl.BlockSpec((2 * F, Fo), lambda i: (0, 0)),
        pl.BlockSpec((1, Fo), lambda i: (0, 0)),
    ]
    if split_out:
        out_specs = pl.BlockSpec((NCORES, R, F2o), lambda i: (0, i, 0))
        out_shape = jax.ShapeDtypeStruct((NCORES, N, F2o), jnp.float32)
    else:
        out_specs = pl.BlockSpec((R, Fo), lambda i: (i, 0))
        out_shape = jax.ShapeDtypeStruct((N, Fo), jnp.float32)

    def body(agg_ref, x_ref, cnt_ref, w_ref, b_ref, o_ref):
        inv = 1.0 / jnp.maximum(cnt_ref[...], 1.0)
        a = jnp.concatenate([agg_ref[0], agg_ref[1]], axis=1) * inv
        xx = jnp.concatenate([x_ref[0], x_ref[1]], axis=1)
        h = jnp.dot(jnp.concatenate([a, xx], axis=1), w_ref[...],
                    preferred_element_type=jnp.float32) + b_ref[...]
        if relu:
            h = jnp.maximum(h, 0.0)
        if split_out:
            o_ref[0] = h[:, :F2o]
            o_ref[1] = h[:, F2o:]
        else:
            o_ref[...] = h

    return pl.pallas_call(body, grid=grid, in_specs=in_specs,
                          out_specs=out_specs, out_shape=out_shape)


@functools.cache
def _pipeline():
    sc64c = _make_sc_agg(64, with_cnt=True)
    sc128 = _make_sc_agg(128, with_cnt=False)
    sc64 = _make_sc_agg(64, with_cnt=False)
    tc1 = _make_tc_layer(64, 256, relu=True, split_out=True)
    tc2 = _make_tc_layer(128, 256, relu=True, split_out=True)
    tc3 = _make_tc_layer(128, 128, relu=True, split_out=True)
    tc4 = _make_tc_layer(64, 128, relu=False, split_out=False)
    return sc64c, sc128, sc64, tc1, tc2, tc3, tc4


def kernel(z, edge_index, Wl1, Wr1, b1, Wl2, Wr2, b2, Wl3, Wr3, b3,
           Wl4, Wr4, b4):
    sc64c, sc128, sc64, tc1, tc2, tc3, tc4 = _pipeline()
    src = edge_index[0]
    dst = edge_index[1]
    z64 = jnp.zeros((NPAD, 64), jnp.float32)
    z128 = jnp.zeros((NPAD, 128), jnp.float32)

    def wcat(Wl, Wr):
        return jnp.concatenate([Wl.T, Wr.T], axis=0)

    x1 = jnp.transpose(z.reshape(N, 2, 64), (1, 0, 2))
    agg1, cnt = sc64c(x1, src, dst, z64)
    cnt2 = cnt.reshape(NPAD, 1)
    x2 = tc1(agg1, x1, cnt2, wcat(Wl1, Wr1), b1.reshape(1, -1))
    agg2 = sc128(x2, src, dst, z128)
    x3 = tc2(agg2, x2, cnt2, wcat(Wl2, Wr2), b2.reshape(1, -1))
    agg3 = sc128(x3, src, dst, z128)
    x4 = tc3(agg3, x3, cnt2, wcat(Wl3, Wr3), b3.reshape(1, -1))
    agg4 = sc64(x4, src, dst, z64)
    return tc4(agg4, x4, cnt2, wcat(Wl4, Wr4), b4.reshape(1, -1))


# speedup vs baseline: 3.7770x; 3.7770x over previous
import functools

import jax
import jax.numpy as jnp
from jax import lax
from jax.experimental import pallas as pl
from jax.experimental.pallas import tpu as pltpu
from jax.experimental.pallas import tpu_sc as plsc

N = 10000
E = 320000
NCORES = 2
NSUB = 16
F2 = 128
NPAD = 10240
RPT = NPAD // NSUB
K = 80
EPT_SPLIT = E // NSUB
EPT_PART = E // (2 * NSUB)


def _make_sc_agg(split_features):
    ept = EPT_SPLIT if split_features else EPT_PART
    nch = ept // K
    scratch = [
        pltpu.VMEM((K,), jnp.int32),
        pltpu.VMEM((K,), jnp.int32),
        pltpu.VMEM((K, F2), jnp.float32),
        pltpu.VMEM_SHARED((NPAD, F2), jnp.float32),
    ]
    mesh = plsc.VectorSubcoreMesh(core_axis_name="c", subcore_axis_name="s")

    def body(x_hbm, src_hbm, dst_hbm, zer_hbm, agg_hbm, idxs, idxd, rows, acc):
        c = lax.axis_index("c")
        s = lax.axis_index("s")
        r0 = s * RPT

        pltpu.sync_copy(zer_hbm.at[pl.ds(r0, RPT)], acc.at[pl.ds(r0, RPT)])
        plsc.subcore_barrier()

        def chunk(j, carry):
            if split_features:
                base = s * ept + j * K
            else:
                base = c * (E // 2) + s * ept + j * K
            pltpu.sync_copy(src_hbm.at[pl.ds(base, K)], idxs)
            pltpu.sync_copy(dst_hbm.at[pl.ds(base, K)], idxd)
            if split_features:
                pltpu.sync_copy(x_hbm.at[c].at[idxs], rows)
            else:
                pltpu.sync_copy(x_hbm.at[idxs], rows)
            pltpu.sync_copy(rows, acc.at[idxd], add=True)
            return carry

        lax.fori_loop(0, nch, chunk, 0)
        plsc.subcore_barrier()
        pltpu.sync_copy(acc.at[pl.ds(r0, RPT)], agg_hbm.at[c].at[pl.ds(r0, RPT)])

    return pl.kernel(
        body,
        out_type=jax.ShapeDtypeStruct((NCORES, NPAD, F2), jnp.float32),
        mesh=mesh, scratch_types=scratch)


def _make_sc_cnt():
    nch = EPT_PART // K
    scratch = [
        pltpu.VMEM((K,), jnp.int32),
        pltpu.VMEM((NPAD,), jnp.float32),
        pltpu.VMEM((NSUB, RPT), jnp.float32),
        pltpu.VMEM((RPT,), jnp.float32),
        pltpu.VMEM_SHARED((NSUB, NPAD), jnp.float32),
    ]
    mesh = plsc.VectorSubcoreMesh(core_axis_name="c", subcore_axis_name="s")

    def body(dst_hbm, cnt_hbm, idxd, hist, cols, red, stage):
        ones16 = jnp.ones((16,), jnp.float32)
        c = lax.axis_index("c")
        s = lax.axis_index("s")
        r0 = s * RPT

        def zero(i, carry):
            hist[pl.ds(i * 16, 16)] = jnp.zeros((16,), jnp.float32)
            return carry
        lax.fori_loop(0, NPAD // 16, zero, 0)

        def chunk(j, carry):
            base = c * (E // 2) + s * EPT_PART + j * K
            pltpu.sync_copy(dst_hbm.at[pl.ds(base, K)], idxd)
            for g in range(K // 16):
                ii = idxd[pl.ds(g * 16, 16)]
                plsc.addupdate_scatter(hist, [ii], ones16)
            return carry
        lax.fori_loop(0, nch, chunk, 0)

        pltpu.sync_copy(hist, stage.at[s])
        plsc.subcore_barrier()
        pltpu.sync_copy(stage.at[:, pl.ds(r0, RPT)], cols)

        def tree(j, carry):
            acc16 = cols[0, pl.ds(j * 16, 16)]
            for i in range(1, NSUB):
                acc16 = acc16 + cols[i, pl.ds(j * 16, 16)]
            red[pl.ds(j * 16, 16)] = acc16
            return carry
        lax.fori_loop(0, RPT // 16, tree, 0)
        pltpu.sync_copy(red, cnt_hbm.at[c].at[pl.ds(r0, RPT)])

    return pl.kernel(
        body,
        out_type=jax.ShapeDtypeStruct((NCORES, NPAD), jnp.float32),
        mesh=mesh, scratch_types=scratch,
        compiler_params=pltpu.CompilerParams(needs_layout_passes=False))


def _make_tc_layer(split_in, Fo, relu, split_out, R=2000):
    F2o = Fo // 2
    grid = (N // R,)
    in_specs = [
        pl.BlockSpec((NCORES, R, F2), lambda i: (0, i, 0)),
        (pl.BlockSpec((NCORES, R, F2), lambda i: (0, i, 0)) if split_in
         else pl.BlockSpec((R, F2), lambda i: (i, 0))),
        pl.BlockSpec((NCORES, R, 1), lambda i: (0, i, 0)),
        pl.BlockSpec(((4 if split_in else 2) * F2, Fo), lambda i: (0, 0)),
        pl.BlockSpec((1, Fo), lambda i: (0, 0)),
    ]
    if split_out:
        out_specs = pl.BlockSpec((NCORES, R, F2o), lambda i: (0, i, 0))
        out_shape = jax.ShapeDtypeStruct((NCORES, N, F2o), jnp.float32)
    else:
        out_specs = pl.BlockSpec((R, Fo), lambda i: (i, 0))
        out_shape = jax.ShapeDtypeStruct((N, Fo), jnp.float32)

    def body(agg_ref, x_ref, cnt_ref, w_ref, b_ref, o_ref):
        inv = 1.0 / jnp.maximum(cnt_ref[0] + cnt_ref[1], 1.0)
        if split_in:
            a = jnp.concatenate([agg_ref[0], agg_ref[1]], axis=1) * inv
            xx = jnp.concatenate([x_ref[0], x_ref[1]], axis=1)
        else:
            a = (agg_ref[0] + agg_ref[1]) * inv
            xx = x_ref[...]
        h = jnp.dot(jnp.concatenate([a, xx], axis=1), w_ref[...],
                    preferred_element_type=jnp.float32) + b_ref[...]
        if relu:
            h = jnp.maximum(h, 0.0)
        if split_out:
            o_ref[0] = h[:, :F2o]
            o_ref[1] = h[:, F2o:]
        else:
            o_ref[...] = h

    return pl.pallas_call(body, grid=grid, in_specs=in_specs,
                          out_specs=out_specs, out_shape=out_shape)


@functools.cache
def _pipeline():
    sc_part = _make_sc_agg(split_features=False)
    sc_split = _make_sc_agg(split_features=True)
    sc_cnt = _make_sc_cnt()
    tc1 = _make_tc_layer(split_in=False, Fo=256, relu=True, split_out=True)
    tc2 = _make_tc_layer(split_in=True, Fo=256, relu=True, split_out=True)
    tc3 = _make_tc_layer(split_in=True, Fo=128, relu=True, split_out=False)
    tc4 = _make_tc_layer(split_in=False, Fo=128, relu=False, split_out=False)
    return sc_part, sc_split, sc_cnt, tc1, tc2, tc3, tc4


def kernel(z, edge_index, Wl1, Wr1, b1, Wl2, Wr2, b2, Wl3, Wr3, b3,
           Wl4, Wr4, b4):
    sc_part, sc_split, sc_cnt, tc1, tc2, tc3, tc4 = _pipeline()
    src = edge_index[0]
    dst = edge_index[1]
    zer = jnp.zeros((NPAD, F2), jnp.float32)

    def wcat(Wl, Wr):
        return jnp.concatenate([Wl.T, Wr.T], axis=0)

    cnt = sc_cnt(dst).reshape(NCORES, NPAD, 1)
    agg1 = sc_part(z, src, dst, zer)
    x2 = tc1(agg1, z, cnt, wcat(Wl1, Wr1), b1.reshape(1, -1))
    agg2 = sc_split(x2, src, dst, zer)
    x3 = tc2(agg2, x2, cnt, wcat(Wl2, Wr2), b2.reshape(1, -1))
    agg3 = sc_split(x3, src, dst, zer)
    x4 = tc3(agg3, x3, cnt, wcat(Wl3, Wr3), b3.reshape(1, -1))
    agg4 = sc_part(x4, src, dst, zer)
    return tc4(agg4, x4, cnt, wcat(Wl4, Wr4), b4.reshape(1, -1))


# speedup vs baseline: 10.3291x; 2.7348x over previous
import functools

import jax
import jax.numpy as jnp
from jax import lax
from jax.experimental import pallas as pl
from jax.experimental.pallas import tpu as pltpu
from jax.experimental.pallas import tpu_sc as plsc

N = 10000
E = 320000
NCORES = 2
NSUB = 16
F2 = 128
NPAD = 10240
RPT = NPAD // NSUB
K = 80
EPT_SPLIT = E // NSUB
EPT_PART = E // (2 * NSUB)


def _make_sc_agg(split_features):
    ept = EPT_SPLIT if split_features else EPT_PART
    nch = ept // K
    NBR = 4
    NBI = 8
    scratch = [
        pltpu.VMEM((NBI, K), jnp.int32),
        pltpu.VMEM((NBI, K), jnp.int32),
        pltpu.VMEM((NBR, K, F2), jnp.float32),
        pltpu.VMEM_SHARED((NPAD, F2), jnp.float32),
    ] + [pltpu.SemaphoreType.DMA] * (2 * NBR + 2 * NBI)
    mesh = plsc.VectorSubcoreMesh(core_axis_name="c", subcore_axis_name="s")

    def body(x_hbm, src_hbm, dst_hbm, zer_hbm, agg_hbm, idxs, idxd, rows,
             acc, *sems):
        gsem = sems[:NBR]
        ssem = sems[NBR:2 * NBR]
        is_sem = sems[2 * NBR:2 * NBR + NBI]
        id_sem = sems[2 * NBR + NBI:]
        c = lax.axis_index("c")
        s = lax.axis_index("s")
        r0 = s * RPT
        if split_features:
            cbase = s * ept
        else:
            cbase = c * (E // 2) + s * ept

        def start_is(j, slot):
            pltpu.async_copy(src_hbm.at[pl.ds(cbase + j * K, K)],
                             idxs.at[slot], is_sem[slot])

        def start_id(j, slot):
            pltpu.async_copy(dst_hbm.at[pl.ds(cbase + j * K, K)],
                             idxd.at[slot], id_sem[slot])

        def wait_is(slot):
            pltpu.make_async_copy(src_hbm.at[pl.ds(cbase, K)],
                                  idxs.at[slot], is_sem[slot]).wait()

        def wait_id(slot):
            pltpu.make_async_copy(dst_hbm.at[pl.ds(cbase, K)],
                                  idxd.at[slot], id_sem[slot]).wait()

        def xsrc(islot):
            tab = x_hbm.at[c] if split_features else x_hbm
            return tab.at[idxs.at[islot]]

        def start_g(islot, rslot):
            pltpu.async_copy(xsrc(islot), rows.at[rslot], gsem[rslot])

        def wait_g(islot, rslot):
            pltpu.make_async_copy(xsrc(islot), rows.at[rslot],
                                  gsem[rslot]).wait()

        def start_s(islot, rslot):
            pltpu.async_copy(rows.at[rslot], acc.at[idxd.at[islot]],
                             ssem[rslot], add=True)

        def wait_s(islot, rslot):
            pltpu.make_async_copy(rows.at[rslot], acc.at[idxd.at[islot]],
                                  ssem[rslot]).wait()

        pltpu.sync_copy(zer_hbm.at[pl.ds(r0, RPT)], acc.at[pl.ds(r0, RPT)])
        for b in range(NBI):
            start_is(b, b)
            start_id(b, b)
        for b in range(3):
            wait_is(b)
            wait_id(b)
            start_g(b, b)
        plsc.subcore_barrier()

        def step(i, carry):
            for b in range(NBI):
                j = i * NBI + b
                rb = b % NBR
                rb3 = (b + 3) % NBR
                ib3 = (b + 3) % NBI
                ib7 = (b + 7) % NBI

                @pl.when(j < nch)
                def _():
                    wait_g(b, rb)
                    start_s(b, rb)

                @pl.when(j + NBI < nch)
                def _():
                    start_is(j + NBI, b)

                @pl.when((j >= 1) & (j - 1 < nch))
                def _():
                    wait_s(ib7, rb3)

                @pl.when((j >= 1) & (j + 7 < nch))
                def _():
                    start_id(j + 7, ib7)

                @pl.when(j + 3 < nch)
                def _():
                    wait_is(ib3)
                    wait_id(ib3)
                    start_g(ib3, rb3)
            return carry

        lax.fori_loop(0, (nch + NBI - 1) // NBI, step, 0)
        if nch % NBI == 0:
            wait_s((nch - 1) % NBI, (nch - 1) % NBR)
        plsc.subcore_barrier()
        pltpu.sync_copy(acc.at[pl.ds(r0, RPT)], agg_hbm.at[c].at[pl.ds(r0, RPT)])

    return pl.kernel(
        body,
        out_type=jax.ShapeDtypeStruct((NCORES, NPAD, F2), jnp.float32),
        mesh=mesh, scratch_types=scratch)


def _make_sc_cnt():
    nch = EPT_PART // K
    scratch = [
        pltpu.VMEM((K,), jnp.int32),
        pltpu.VMEM((NPAD,), jnp.float32),
        pltpu.VMEM((NSUB, RPT), jnp.float32),
        pltpu.VMEM((RPT,), jnp.float32),
        pltpu.VMEM_SHARED((NSUB, NPAD), jnp.float32),
    ]
    mesh = plsc.VectorSubcoreMesh(core_axis_name="c", subcore_axis_name="s")

    def body(dst_hbm, cnt_hbm, idxd, hist, cols, red, stage):
        ones16 = jnp.ones((16,), jnp.float32)
        c = lax.axis_index("c")
        s = lax.axis_index("s")
        r0 = s * RPT

        def zero(i, carry):
            hist[pl.ds(i * 16, 16)] = jnp.zeros((16,), jnp.float32)
            return carry
        lax.fori_loop(0, NPAD // 16, zero, 0)

        def chunk(j, carry):
            base = c * (E // 2) + s * EPT_PART + j * K
            pltpu.sync_copy(dst_hbm.at[pl.ds(base, K)], idxd)
            for g in range(K // 16):
                ii = idxd[pl.ds(g * 16, 16)]
                plsc.addupdate_scatter(hist, [ii], ones16)
            return carry
        lax.fori_loop(0, nch, chunk, 0)

        pltpu.sync_copy(hist, stage.at[s])
        plsc.subcore_barrier()
        pltpu.sync_copy(stage.at[:, pl.ds(r0, RPT)], cols)

        def tree(j, carry):
            acc16 = cols[0, pl.ds(j * 16, 16)]
            for i in range(1, NSUB):
                acc16 = acc16 + cols[i, pl.ds(j * 16, 16)]
            red[pl.ds(j * 16, 16)] = acc16
            return carry
        lax.fori_loop(0, RPT // 16, tree, 0)
        pltpu.sync_copy(red, cnt_hbm.at[c].at[pl.ds(r0, RPT)])

    return pl.kernel(
        body,
        out_type=jax.ShapeDtypeStruct((NCORES, NPAD), jnp.float32),
        mesh=mesh, scratch_types=scratch,
        compiler_params=pltpu.CompilerParams(needs_layout_passes=False))


def _make_tc_layer(split_in, Fo, relu, split_out, R=2000):
    F2o = Fo // 2
    grid = (N // R,)
    in_specs = [
        pl.BlockSpec((NCORES, R, F2), lambda i: (0, i, 0)),
        (pl.BlockSpec((NCORES, R, F2), lambda i: (0, i, 0)) if split_in
         else pl.BlockSpec((R, F2), lambda i: (i, 0))),
        pl.BlockSpec((NCORES, R, 1), lambda i: (0, i, 0)),
        pl.BlockSpec(((4 if split_in else 2) * F2, Fo), lambda i: (0, 0)),
        pl.BlockSpec((1, Fo), lambda i: (0, 0)),
    ]
    if split_out:
        out_specs = pl.BlockSpec((NCORES, R, F2o), lambda i: (0, i, 0))
        out_shape = jax.ShapeDtypeStruct((NCORES, N, F2o), jnp.float32)
    else:
        out_specs = pl.BlockSpec((R, Fo), lambda i: (i, 0))
        out_shape = jax.ShapeDtypeStruct((N, Fo), jnp.float32)

    def body(agg_ref, x_ref, cnt_ref, w_ref, b_ref, o_ref):
        inv = 1.0 / jnp.maximum(cnt_ref[0] + cnt_ref[1], 1.0)
        if split_in:
            a = jnp.concatenate([agg_ref[0], agg_ref[1]], axis=1) * inv
            xx = jnp.concatenate([x_ref[0], x_ref[1]], axis=1)
        else:
            a = (agg_ref[0] + agg_ref[1]) * inv
            xx = x_ref[...]
        h = jnp.dot(jnp.concatenate([a, xx], axis=1), w_ref[...],
                    preferred_element_type=jnp.float32) + b_ref[...]
        if relu:
            h = jnp.maximum(h, 0.0)
        if split_out:
            o_ref[0] = h[:, :F2o]
            o_ref[1] = h[:, F2o:]
        else:
            o_ref[...] = h

    return pl.pallas_call(body, grid=grid, in_specs=in_specs,
                          out_specs=out_specs, out_shape=out_shape)


@functools.cache
def _pipeline():
    sc_part = _make_sc_agg(split_features=False)
    sc_split = _make_sc_agg(split_features=True)
    sc_cnt = _make_sc_cnt()
    tc1 = _make_tc_layer(split_in=False, Fo=256, relu=True, split_out=True)
    tc2 = _make_tc_layer(split_in=True, Fo=256, relu=True, split_out=True)
    tc3 = _make_tc_layer(split_in=True, Fo=128, relu=True, split_out=False)
    tc4 = _make_tc_layer(split_in=False, Fo=128, relu=False, split_out=False)
    return sc_part, sc_split, sc_cnt, tc1, tc2, tc3, tc4


def kernel(z, edge_index, Wl1, Wr1, b1, Wl2, Wr2, b2, Wl3, Wr3, b3,
           Wl4, Wr4, b4):
    sc_part, sc_split, sc_cnt, tc1, tc2, tc3, tc4 = _pipeline()
    src = edge_index[0]
    dst = edge_index[1]
    zer = jnp.zeros((NPAD, F2), jnp.float32)

    def wcat(Wl, Wr):
        return jnp.concatenate([Wl.T, Wr.T], axis=0)

    cnt = sc_cnt(dst).reshape(NCORES, NPAD, 1)
    agg1 = sc_part(z, src, dst, zer)
    x2 = tc1(agg1, z, cnt, wcat(Wl1, Wr1), b1.reshape(1, -1))
    agg2 = sc_split(x2, src, dst, zer)
    x3 = tc2(agg2, x2, cnt, wcat(Wl2, Wr2), b2.reshape(1, -1))
    agg3 = sc_split(x3, src, dst, zer)
    x4 = tc3(agg3, x3, cnt, wcat(Wl3, Wr3), b3.reshape(1, -1))
    agg4 = sc_part(x4, src, dst, zer)
    return tc4(agg4, x4, cnt, wcat(Wl4, Wr4), b4.reshape(1, -1))
